# Initial kernel scaffold; baseline (speedup 1.0000x reference)
import jax
import jax.numpy as jnp
from jax.experimental import pallas as pl


def kernel(x, edge_index, W1_l, b1, W1_r, W2_l, b2, W2_r):
    raise NotImplementedError("write your pallas kernel here")


# speedup vs baseline: 3.6786x; 3.6786x over previous
import functools

import jax
import jax.numpy as jnp
from jax import lax
from jax.experimental import pallas as pl
from jax.experimental.pallas import tpu as pltpu
from jax.experimental.pallas import tpu_sc as plsc

N_NODES = 10000
D = 128
N_EDGES = 320000

NC = 2
NS = 16
NW = NC * NS

C = 128
CPT = 80
EPT = C * CPT
EPAD = NW * EPT

NPAD = 10240
RPT = NPAD // NS
ROWS_PER_TILE = NPAD // NS


def _make_seg(with_counts):
    mesh = plsc.VectorSubcoreMesh(core_axis_name="c", subcore_axis_name="s")
    out_type = [jax.ShapeDtypeStruct((NC * NPAD, D), jnp.float32)]
    if with_counts:
        out_type.append(jax.ShapeDtypeStruct((NW, NPAD), jnp.float32))

    scratch = [
        pltpu.VMEM((C,), jnp.int32),
        pltpu.VMEM((C,), jnp.int32),
        pltpu.VMEM((C,), jnp.int32),
        pltpu.VMEM((C,), jnp.int32),
        pltpu.VMEM((C, D), jnp.float32),
        pltpu.VMEM((C, D), jnp.float32),
        pltpu.VMEM((16, D), jnp.float32),
        pltpu.VMEM_SHARED((NPAD, D), jnp.float32),
        pltpu.SemaphoreType.DMA,
        pltpu.SemaphoreType.DMA,
    ]
    if with_counts:
        scratch.append(pltpu.VMEM((NPAD,), jnp.float32))

    def body(vals_hbm, srcp_hbm, dstp_hbm, *rest):
        if with_counts:
            p_hbm, cnt_hbm = rest[0], rest[1]
            rest = rest[2:]
        else:
            p_hbm = rest[0]
            rest = rest[1:]
        sidx = rest[0:2]
        didx = rest[2:4]
        rows = rest[4:6]
        zrow = rest[6]
        acc = rest[7]
        sems = rest[8:10]
        cntv = rest[10] if with_counts else None

        cid = lax.axis_index("c")
        sid = lax.axis_index("s")
        wid = sid * NC + cid
        ebase = wid * EPT

        def zb(i, carry):
            for j in range(D // 16):
                zrow[i, pl.ds(j * 16, 16)] = jnp.zeros((16,), jnp.float32)
            return carry

        lax.fori_loop(0, 16, zb, 0)

        nrep = ROWS_PER_TILE // 16
        def zc(k, carry):
            pltpu.sync_copy(zrow, acc.at[pl.ds(sid * ROWS_PER_TILE + k * 16, 16)])
            return carry

        lax.fori_loop(0, nrep, zc, 0)

        if with_counts:
            def zcnt(i, carry):
                cntv[pl.ds(i * 16, 16)] = jnp.zeros((16,), jnp.float32)
                return carry

            lax.fori_loop(0, NPAD // 16, zcnt, 0)

        plsc.subcore_barrier()

        ones16 = jnp.ones((16,), jnp.float32)

        def issue(b, ci):
            off = ebase + ci * C
            pltpu.sync_copy(srcp_hbm.at[pl.ds(off, C)], sidx[b])
            pltpu.sync_copy(dstp_hbm.at[pl.ds(off, C)], didx[b])
            pltpu.async_copy(vals_hbm.at[sidx[b]], rows[b], sems[b])

        def drain(b):
            pltpu.make_async_copy(vals_hbm.at[sidx[b]], rows[b], sems[b]).wait()
            pltpu.sync_copy(rows[b], acc.at[didx[b]], add=True)
            if with_counts:
                for j in range(C // 16):
                    dvec = didx[b][pl.ds(j * 16, 16)]
                    plsc.addupdate_scatter(cntv, [dvec], ones16)

        issue(0, 0)
        issue(1, 1)

        def step(g, carry):
            for b in range(2):
                drain(b)
                issue(b, g * 2 + b + 2)
            return carry

        lax.fori_loop(0, CPT // 2 - 1, step, 0)
        drain(0)
        drain(1)

        plsc.subcore_barrier()

        rbase = sid * ROWS_PER_TILE
        pltpu.sync_copy(
            acc.at[pl.ds(rbase, ROWS_PER_TILE)],
            p_hbm.at[pl.ds(cid * NPAD + rbase, ROWS_PER_TILE)],
        )
        if with_counts:
            pltpu.sync_copy(cntv, cnt_hbm.at[wid])

    return pl.kernel(
        body,
        out_type=tuple(out_type),
        mesh=mesh,
        scratch_types=scratch,
        compiler_params=pltpu.CompilerParams(needs_layout_passes=False),
    )


_SEG_COUNTS = _make_seg(True)
_SEG = _make_seg(False)

BLK = 1024


def _combine_body(p0_ref, p1_ref, cnt_ref, v_ref, wl_ref, wr_ref, b_ref, o_ref, *, act):
    cnt = jnp.sum(cnt_ref[...], axis=0)
    recip = 1.0 / jnp.maximum(cnt, 1.0)
    agg = (p0_ref[...] + p1_ref[...]) * recip[:, None]
    r = (
        jnp.dot(agg, wl_ref[...], preferred_element_type=jnp.float32)
        + jnp.dot(v_ref[...], wr_ref[...], preferred_element_type=jnp.float32)
        + b_ref[...]
    )
    o_ref[...] = act(r)


def _make_combine(act):
    return pl.pallas_call(
        functools.partial(_combine_body, act=act),
        grid=(NPAD // BLK,),
        in_specs=[
            pl.BlockSpec((BLK, D), lambda i: (i, 0)),
            pl.BlockSpec((BLK, D), lambda i: (i + NPAD // BLK, 0)),
            pl.BlockSpec((NW, BLK), lambda i: (0, i)),
            pl.BlockSpec((BLK, D), lambda i: (i, 0)),
            pl.BlockSpec((D, D), lambda i: (0, 0)),
            pl.BlockSpec((D, D), lambda i: (0, 0)),
            pl.BlockSpec((1, D), lambda i: (0, 0)),
        ],
        out_specs=pl.BlockSpec((BLK, D), lambda i: (i, 0)),
        out_shape=jax.ShapeDtypeStruct((NPAD, D), jnp.float32),
    )


_COMBINE_RELU = _make_combine(jax.nn.relu)
_COMBINE_SIGMOID = _make_combine(jax.nn.sigmoid)


def kernel(x, edge_index, W1_l, b1, W1_r, W2_l, b2, W2_r):
    src = edge_index[0].astype(jnp.int32)
    dst = edge_index[1].astype(jnp.int32)
    pad_e = EPAD - N_EDGES
    srcp = jnp.concatenate([src, jnp.zeros((pad_e,), jnp.int32)])
    dstp = jnp.concatenate([dst, jnp.full((pad_e,), NPAD - 1, jnp.int32)])
    xp = jnp.concatenate(
        [x.astype(jnp.float32), jnp.zeros((NPAD - N_NODES, D), jnp.float32)]
    )

    p1, cnt = _SEG_COUNTS(xp, srcp, dstp)
    h = _COMBINE_RELU(p1, p1, cnt, xp, W1_l.T, W1_r.T, b1.reshape(1, D))
    p2 = _SEG(h, srcp, dstp)
    if isinstance(p2, (list, tuple)):
        p2 = p2[0]
    out = _COMBINE_SIGMOID(p2, p2, cnt, h, W2_l.T, W2_r.T, b2.reshape(1, D))
    return out[:N_NODES]


# speedup vs baseline: 3.9241x; 1.0667x over previous
import functools

import jax
import jax.numpy as jnp
from jax import lax
from jax.experimental import pallas as pl
from jax.experimental.pallas import tpu as pltpu
from jax.experimental.pallas import tpu_sc as plsc

N_NODES = 10000
D = 128
N_EDGES = 320000

NC = 2
NS = 16
NW = NC * NS

C = 128
CPT0 = 116
CPT1 = 44
NCHUNKS = NS * (CPT0 + CPT1)
EPAD = NCHUNKS * C

NPAD = 10240
RPT = NPAD // NS
ROWS_PER_TILE = NPAD // NS


def _make_seg(with_counts):
    mesh = plsc.VectorSubcoreMesh(core_axis_name="c", subcore_axis_name="s")
    out_type = [jax.ShapeDtypeStruct((NC * NPAD, D), jnp.float32)]
    if with_counts:
        out_type.append(jax.ShapeDtypeStruct((NW, NPAD), jnp.float32))

    scratch = [
        pltpu.VMEM((C,), jnp.int32),
        pltpu.VMEM((C,), jnp.int32),
        pltpu.VMEM((C,), jnp.int32),
        pltpu.VMEM((C,), jnp.int32),
        pltpu.VMEM((C, D), jnp.float32),
        pltpu.VMEM((C, D), jnp.float32),
        pltpu.VMEM((16, D), jnp.float32),
        pltpu.VMEM_SHARED((NPAD, D), jnp.float32),
        pltpu.SemaphoreType.DMA,
        pltpu.SemaphoreType.DMA,
    ]
    if with_counts:
        scratch.append(pltpu.VMEM((NPAD,), jnp.float32))

    def body(vals_hbm, srcp_hbm, dstp_hbm, *rest):
        if with_counts:
            p_hbm, cnt_hbm = rest[0], rest[1]
            rest = rest[2:]
        else:
            p_hbm = rest[0]
            rest = rest[1:]
        sidx = rest[0:2]
        didx = rest[2:4]
        rows = rest[4:6]
        zrow = rest[6]
        acc = rest[7]
        sems = rest[8:10]
        cntv = rest[10] if with_counts else None

        cid = lax.axis_index("c")
        sid = lax.axis_index("s")
        wid = sid * NC + cid
        ebase = jnp.where(
            cid == 0,
            sid * (CPT0 * C),
            NS * (CPT0 * C) + sid * (CPT1 * C),
        )
        nsteps = jnp.where(cid == 0, CPT0 // 2 - 1, CPT1 // 2 - 1)

        def zb(i, carry):
            for j in range(D // 16):
                zrow[i, pl.ds(j * 16, 16)] = jnp.zeros((16,), jnp.float32)
            return carry

        lax.fori_loop(0, 16, zb, 0)

        nrep = ROWS_PER_TILE // 16
        def zc(k, carry):
            pltpu.sync_copy(zrow, acc.at[pl.ds(sid * ROWS_PER_TILE + k * 16, 16)])
            return carry

        lax.fori_loop(0, nrep, zc, 0)

        if with_counts:
            def zcnt(i, carry):
                cntv[pl.ds(i * 16, 16)] = jnp.zeros((16,), jnp.float32)
                return carry

            lax.fori_loop(0, NPAD // 16, zcnt, 0)

        plsc.subcore_barrier()

        ones16 = jnp.ones((16,), jnp.float32)

        def issue(b, ci):
            off = ebase + ci * C
            pltpu.sync_copy(srcp_hbm.at[pl.ds(off, C)], sidx[b])
            pltpu.sync_copy(dstp_hbm.at[pl.ds(off, C)], didx[b])
            pltpu.async_copy(vals_hbm.at[sidx[b]], rows[b], sems[b])

        def drain(b):
            pltpu.make_async_copy(vals_hbm.at[sidx[b]], rows[b], sems[b]).wait()
            pltpu.sync_copy(rows[b], acc.at[didx[b]], add=True)
            if with_counts:
                for j in range(C // 16):
                    dvec = didx[b][pl.ds(j * 16, 16)]
                    plsc.addupdate_scatter(cntv, [dvec], ones16)

        issue(0, 0)
        issue(1, 1)

        def step(g, carry):
            for b in range(2):
                drain(b)
                issue(b, g * 2 + b + 2)
            return carry

        lax.fori_loop(0, nsteps, step, 0)
        drain(0)
        drain(1)

        plsc.subcore_barrier()

        rbase = sid * ROWS_PER_TILE
        pltpu.sync_copy(
            acc.at[pl.ds(rbase, ROWS_PER_TILE)],
            p_hbm.at[pl.ds(cid * NPAD + rbase, ROWS_PER_TILE)],
        )
        if with_counts:
            pltpu.sync_copy(cntv, cnt_hbm.at[wid])

    return pl.kernel(
        body,
        out_type=tuple(out_type),
        mesh=mesh,
        scratch_types=scratch,
        compiler_params=pltpu.CompilerParams(needs_layout_passes=False),
    )


_SEG_COUNTS = _make_seg(True)
_SEG = _make_seg(False)

BLK = 1024


def _combine_body(p0_ref, p1_ref, cnt_ref, v_ref, wl_ref, wr_ref, b_ref, o_ref, *, act):
    cnt = jnp.sum(cnt_ref[...], axis=0)
    recip = 1.0 / jnp.maximum(cnt, 1.0)
    agg = (p0_ref[...] + p1_ref[...]) * recip[:, None]
    r = (
        jnp.dot(agg, wl_ref[...], preferred_element_type=jnp.float32)
        + jnp.dot(v_ref[...], wr_ref[...], preferred_element_type=jnp.float32)
        + b_ref[...]
    )
    o_ref[...] = act(r)


def _make_combine(act):
    return pl.pallas_call(
        functools.partial(_combine_body, act=act),
        grid=(NPAD // BLK,),
        in_specs=[
            pl.BlockSpec((BLK, D), lambda i: (i, 0)),
            pl.BlockSpec((BLK, D), lambda i: (i + NPAD // BLK, 0)),
            pl.BlockSpec((NW, BLK), lambda i: (0, i)),
            pl.BlockSpec((BLK, D), lambda i: (i, 0)),
            pl.BlockSpec((D, D), lambda i: (0, 0)),
            pl.BlockSpec((D, D), lambda i: (0, 0)),
            pl.BlockSpec((1, D), lambda i: (0, 0)),
        ],
        out_specs=pl.BlockSpec((BLK, D), lambda i: (i, 0)),
        out_shape=jax.ShapeDtypeStruct((NPAD, D), jnp.float32),
    )


_COMBINE_RELU = _make_combine(jax.nn.relu)
_COMBINE_SIGMOID = _make_combine(jax.nn.sigmoid)


def kernel(x, edge_index, W1_l, b1, W1_r, W2_l, b2, W2_r):
    src = edge_index[0].astype(jnp.int32)
    dst = edge_index[1].astype(jnp.int32)
    pad_e = EPAD - N_EDGES
    srcp = jnp.concatenate([src, jnp.zeros((pad_e,), jnp.int32)])
    dstp = jnp.concatenate([dst, jnp.full((pad_e,), NPAD - 1, jnp.int32)])
    xp = jnp.concatenate(
        [x.astype(jnp.float32), jnp.zeros((NPAD - N_NODES, D), jnp.float32)]
    )

    p1, cnt = _SEG_COUNTS(xp, srcp, dstp)
    h = _COMBINE_RELU(p1, p1, cnt, xp, W1_l.T, W1_r.T, b1.reshape(1, D))
    p2 = _SEG(h, srcp, dstp)
    if isinstance(p2, (list, tuple)):
        p2 = p2[0]
    out = _COMBINE_SIGMOID(p2, p2, cnt, h, W2_l.T, W2_r.T, b2.reshape(1, D))
    return out[:N_NODES]


# speedup vs baseline: 3.9259x; 1.0005x over previous
import functools

import jax
import jax.numpy as jnp
from jax import lax
from jax.experimental import pallas as pl
from jax.experimental.pallas import tpu as pltpu
from jax.experimental.pallas import tpu_sc as plsc

N_NODES = 10000
D = 128
N_EDGES = 320000

NC = 2
NS = 16
NW = NC * NS

C = 128
CPT0 = 116
CPT1 = 44
NCHUNKS = NS * (CPT0 + CPT1)
EPAD = NCHUNKS * C

NPAD = 10240
RPT = NPAD // NS
ROWS_PER_TILE = NPAD // NS


def _make_seg(with_counts):
    mesh = plsc.VectorSubcoreMesh(core_axis_name="c", subcore_axis_name="s")
    out_type = [jax.ShapeDtypeStruct((NC * NPAD, D), jnp.float32)]
    if with_counts:
        out_type.append(jax.ShapeDtypeStruct((NW, NPAD), jnp.float32))

    scratch = [
        pltpu.VMEM((C,), jnp.int32),
        pltpu.VMEM((C,), jnp.int32),
        pltpu.VMEM((C,), jnp.int32),
        pltpu.VMEM((C,), jnp.int32),
        pltpu.VMEM((C, D), jnp.float32),
        pltpu.VMEM((C, D), jnp.float32),
        pltpu.VMEM((16, D), jnp.float32),
        pltpu.VMEM_SHARED((NPAD, D), jnp.float32),
        pltpu.SemaphoreType.DMA,
        pltpu.SemaphoreType.DMA,
    ]
    if with_counts:
        scratch.append(pltpu.VMEM((NPAD,), jnp.float32))

    def body(vals_hbm, srcp_hbm, dstp_hbm, *rest):
        if with_counts:
            p_hbm, cnt_hbm = rest[0], rest[1]
            rest = rest[2:]
        else:
            p_hbm = rest[0]
            rest = rest[1:]
        sidx = rest[0:2]
        didx = rest[2:4]
        rows = rest[4:6]
        zrow = rest[6]
        acc = rest[7]
        sems = rest[8:10]
        cntv = rest[10] if with_counts else None

        cid = lax.axis_index("c")
        sid = lax.axis_index("s")
        wid = sid * NC + cid
        ebase = jnp.where(
            cid == 0,
            sid * (CPT0 * C),
            NS * (CPT0 * C) + sid * (CPT1 * C),
        )
        nsteps = jnp.where(cid == 0, CPT0 // 2 - 1, CPT1 // 2 - 1)

        def zb(i, carry):
            for j in range(D // 16):
                zrow[i, pl.ds(j * 16, 16)] = jnp.zeros((16,), jnp.float32)
            return carry

        with jax.named_scope("zinit"):
            lax.fori_loop(0, 16, zb, 0)

        nrep = ROWS_PER_TILE // 16
        def zc(k, carry):
            pltpu.sync_copy(zrow, acc.at[pl.ds(sid * ROWS_PER_TILE + k * 16, 16)])
            return carry

        with jax.named_scope("zcopy"):
            lax.fori_loop(0, nrep, zc, 0)

        if with_counts:
            def zcnt(i, carry):
                cntv[pl.ds(i * 16, 16)] = jnp.zeros((16,), jnp.float32)
                return carry

            with jax.named_scope("zcnt"):
                lax.fori_loop(0, NPAD // 16, zcnt, 0)

        plsc.subcore_barrier()

        ones16 = jnp.ones((16,), jnp.float32)

        def issue(b, ci):
            off = ebase + ci * C
            pltpu.sync_copy(srcp_hbm.at[pl.ds(off, C)], sidx[b])
            pltpu.sync_copy(dstp_hbm.at[pl.ds(off, C)], didx[b])
            pltpu.async_copy(vals_hbm.at[sidx[b]], rows[b], sems[b])

        def drain(b):
            pltpu.make_async_copy(vals_hbm.at[sidx[b]], rows[b], sems[b]).wait()
            pltpu.sync_copy(rows[b], acc.at[didx[b]], add=True)
            if with_counts:
                for j in range(C // 16):
                    dvec = didx[b][pl.ds(j * 16, 16)]
                    plsc.addupdate_scatter(cntv, [dvec], ones16)

        with jax.named_scope("mainloop"):
            issue(0, 0)
            issue(1, 1)

            def step(g, carry):
                for b in range(2):
                    drain(b)
                    issue(b, g * 2 + b + 2)
                return carry

            lax.fori_loop(0, nsteps, step, 0)
            drain(0)
            drain(1)

        with jax.named_scope("outbar"):
            plsc.subcore_barrier()

        with jax.named_scope("outcopy"):
            rbase = sid * ROWS_PER_TILE
            pltpu.sync_copy(
                acc.at[pl.ds(rbase, ROWS_PER_TILE)],
                p_hbm.at[pl.ds(cid * NPAD + rbase, ROWS_PER_TILE)],
            )
            if with_counts:
                pltpu.sync_copy(cntv, cnt_hbm.at[wid])

    return pl.kernel(
        body,
        out_type=tuple(out_type),
        mesh=mesh,
        scratch_types=scratch,
        compiler_params=pltpu.CompilerParams(needs_layout_passes=False),
    )


_SEG_COUNTS = _make_seg(True)
_SEG = _make_seg(False)

BLK = 1024


def _combine_body(p0_ref, p1_ref, cnt_ref, v_ref, wl_ref, wr_ref, b_ref, o_ref, *, act):
    cnt = jnp.sum(cnt_ref[...], axis=0)
    recip = 1.0 / jnp.maximum(cnt, 1.0)
    agg = (p0_ref[...] + p1_ref[...]) * recip[:, None]
    r = (
        jnp.dot(agg, wl_ref[...], preferred_element_type=jnp.float32)
        + jnp.dot(v_ref[...], wr_ref[...], preferred_element_type=jnp.float32)
        + b_ref[...]
    )
    o_ref[...] = act(r)


def _make_combine(act):
    return pl.pallas_call(
        functools.partial(_combine_body, act=act),
        grid=(NPAD // BLK,),
        in_specs=[
            pl.BlockSpec((BLK, D), lambda i: (i, 0)),
            pl.BlockSpec((BLK, D), lambda i: (i + NPAD // BLK, 0)),
            pl.BlockSpec((NW, BLK), lambda i: (0, i)),
            pl.BlockSpec((BLK, D), lambda i: (i, 0)),
            pl.BlockSpec((D, D), lambda i: (0, 0)),
            pl.BlockSpec((D, D), lambda i: (0, 0)),
            pl.BlockSpec((1, D), lambda i: (0, 0)),
        ],
        out_specs=pl.BlockSpec((BLK, D), lambda i: (i, 0)),
        out_shape=jax.ShapeDtypeStruct((NPAD, D), jnp.float32),
    )


_COMBINE_RELU = _make_combine(jax.nn.relu)
_COMBINE_SIGMOID = _make_combine(jax.nn.sigmoid)


def kernel(x, edge_index, W1_l, b1, W1_r, W2_l, b2, W2_r):
    src = edge_index[0].astype(jnp.int32)
    dst = edge_index[1].astype(jnp.int32)
    pad_e = EPAD - N_EDGES
    srcp = jnp.concatenate([src, jnp.zeros((pad_e,), jnp.int32)])
    dstp = jnp.concatenate([dst, jnp.full((pad_e,), NPAD - 1, jnp.int32)])
    xp = jnp.concatenate(
        [x.astype(jnp.float32), jnp.zeros((NPAD - N_NODES, D), jnp.float32)]
    )

    p1, cnt = _SEG_COUNTS(xp, srcp, dstp)
    h = _COMBINE_RELU(p1, p1, cnt, xp, W1_l.T, W1_r.T, b1.reshape(1, D))
    p2 = _SEG(h, srcp, dstp)
    if isinstance(p2, (list, tuple)):
        p2 = p2[0]
    out = _COMBINE_SIGMOID(p2, p2, cnt, h, W2_l.T, W2_r.T, b2.reshape(1, D))
    return out[:N_NODES]


# speedup vs baseline: 4.1400x; 1.0545x over previous
import functools

import jax
import jax.numpy as jnp
from jax import lax
from jax.experimental import pallas as pl
from jax.experimental.pallas import tpu as pltpu
from jax.experimental.pallas import tpu_sc as plsc

N_NODES = 10000
D = 128
N_EDGES = 320000

NC = 2
NS = 16
NW = NC * NS

C = 128
CPT0 = 136
CPT1 = 24
NCHUNKS = NS * (CPT0 + CPT1)
EPAD = NCHUNKS * C

NPAD = 10240
RPT = NPAD // NS
ROWS_PER_TILE = NPAD // NS


def _make_seg(with_counts):
    mesh = plsc.VectorSubcoreMesh(core_axis_name="c", subcore_axis_name="s")
    out_type = [jax.ShapeDtypeStruct((NC * NPAD, D), jnp.float32)]
    if with_counts:
        out_type.append(jax.ShapeDtypeStruct((NW, NPAD), jnp.float32))

    scratch = [
        pltpu.VMEM((C,), jnp.int32),
        pltpu.VMEM((C,), jnp.int32),
        pltpu.VMEM((C,), jnp.int32),
        pltpu.VMEM((C,), jnp.int32),
        pltpu.VMEM((C, D), jnp.float32),
        pltpu.VMEM((C, D), jnp.float32),
        pltpu.VMEM((16, D), jnp.float32),
        pltpu.VMEM_SHARED((NPAD, D), jnp.float32),
        pltpu.SemaphoreType.DMA,
        pltpu.SemaphoreType.DMA,
    ]
    if with_counts:
        scratch.append(pltpu.VMEM((NPAD,), jnp.float32))

    def body(vals_hbm, srcp_hbm, dstp_hbm, *rest):
        if with_counts:
            p_hbm, cnt_hbm = rest[0], rest[1]
            rest = rest[2:]
        else:
            p_hbm = rest[0]
            rest = rest[1:]
        sidx = rest[0:2]
        didx = rest[2:4]
        rows = rest[4:6]
        zrow = rest[6]
        acc = rest[7]
        sems = rest[8:10]
        cntv = rest[10] if with_counts else None

        cid = lax.axis_index("c")
        sid = lax.axis_index("s")
        wid = sid * NC + cid
        ebase = jnp.where(
            cid == 0,
            sid * (CPT0 * C),
            NS * (CPT0 * C) + sid * (CPT1 * C),
        )
        nsteps = jnp.where(cid == 0, CPT0 // 2 - 1, CPT1 // 2 - 1)

        def zb(i, carry):
            for j in range(D // 16):
                zrow[i, pl.ds(j * 16, 16)] = jnp.zeros((16,), jnp.float32)
            return carry

        with jax.named_scope("zinit"):
            lax.fori_loop(0, 16, zb, 0)

        nrep = ROWS_PER_TILE // 16
        def zc(k, carry):
            pltpu.sync_copy(zrow, acc.at[pl.ds(sid * ROWS_PER_TILE + k * 16, 16)])
            return carry

        with jax.named_scope("zcopy"):
            lax.fori_loop(0, nrep, zc, 0)

        if with_counts:
            def zcnt(i, carry):
                cntv[pl.ds(i * 16, 16)] = jnp.zeros((16,), jnp.float32)
                return carry

            with jax.named_scope("zcnt"):
                lax.fori_loop(0, NPAD // 16, zcnt, 0)

        plsc.subcore_barrier()

        ones16 = jnp.ones((16,), jnp.float32)

        def issue(b, ci):
            off = ebase + ci * C
            pltpu.sync_copy(srcp_hbm.at[pl.ds(off, C)], sidx[b])
            pltpu.sync_copy(dstp_hbm.at[pl.ds(off, C)], didx[b])
            pltpu.async_copy(vals_hbm.at[sidx[b]], rows[b], sems[b])

        def drain(b):
            pltpu.make_async_copy(vals_hbm.at[sidx[b]], rows[b], sems[b]).wait()
            pltpu.sync_copy(rows[b], acc.at[didx[b]], add=True)
            if with_counts:
                for j in range(C // 16):
                    dvec = didx[b][pl.ds(j * 16, 16)]
                    plsc.addupdate_scatter(cntv, [dvec], ones16)

        with jax.named_scope("mainloop"):
            issue(0, 0)
            issue(1, 1)

            def step(g, carry):
                for b in range(2):
                    drain(b)
                    issue(b, g * 2 + b + 2)
                return carry

            lax.fori_loop(0, nsteps, step, 0)
            drain(0)
            drain(1)

        with jax.named_scope("outbar"):
            plsc.subcore_barrier()

        with jax.named_scope("outcopy"):
            rbase = sid * ROWS_PER_TILE
            pltpu.sync_copy(
                acc.at[pl.ds(rbase, ROWS_PER_TILE)],
                p_hbm.at[pl.ds(cid * NPAD + rbase, ROWS_PER_TILE)],
            )
            if with_counts:
                pltpu.sync_copy(cntv, cnt_hbm.at[wid])

    return pl.kernel(
        body,
        out_type=tuple(out_type),
        mesh=mesh,
        scratch_types=scratch,
        compiler_params=pltpu.CompilerParams(needs_layout_passes=False),
    )


_SEG_COUNTS = _make_seg(True)
_SEG = _make_seg(False)

BLK = 1024


def _combine_body(p0_ref, p1_ref, cnt_ref, v_ref, wl_ref, wr_ref, b_ref, o_ref, *, act):
    cnt = jnp.sum(cnt_ref[...], axis=0)
    recip = 1.0 / jnp.maximum(cnt, 1.0)
    agg = (p0_ref[...] + p1_ref[...]) * recip[:, None]
    r = (
        jnp.dot(agg, wl_ref[...], preferred_element_type=jnp.float32)
        + jnp.dot(v_ref[...], wr_ref[...], preferred_element_type=jnp.float32)
        + b_ref[...]
    )
    o_ref[...] = act(r)


def _make_combine(act):
    return pl.pallas_call(
        functools.partial(_combine_body, act=act),
        grid=(NPAD // BLK,),
        in_specs=[
            pl.BlockSpec((BLK, D), lambda i: (i, 0)),
            pl.BlockSpec((BLK, D), lambda i: (i + NPAD // BLK, 0)),
            pl.BlockSpec((NW, BLK), lambda i: (0, i)),
            pl.BlockSpec((BLK, D), lambda i: (i, 0)),
            pl.BlockSpec((D, D), lambda i: (0, 0)),
            pl.BlockSpec((D, D), lambda i: (0, 0)),
            pl.BlockSpec((1, D), lambda i: (0, 0)),
        ],
        out_specs=pl.BlockSpec((BLK, D), lambda i: (i, 0)),
        out_shape=jax.ShapeDtypeStruct((NPAD, D), jnp.float32),
    )


_COMBINE_RELU = _make_combine(jax.nn.relu)
_COMBINE_SIGMOID = _make_combine(jax.nn.sigmoid)


def kernel(x, edge_index, W1_l, b1, W1_r, W2_l, b2, W2_r):
    src = edge_index[0].astype(jnp.int32)
    dst = edge_index[1].astype(jnp.int32)
    pad_e = EPAD - N_EDGES
    srcp = jnp.concatenate([src, jnp.zeros((pad_e,), jnp.int32)])
    dstp = jnp.concatenate([dst, jnp.full((pad_e,), NPAD - 1, jnp.int32)])
    xp = jnp.concatenate(
        [x.astype(jnp.float32), jnp.zeros((NPAD - N_NODES, D), jnp.float32)]
    )

    p1, cnt = _SEG_COUNTS(xp, srcp, dstp)
    h = _COMBINE_RELU(p1, p1, cnt, xp, W1_l.T, W1_r.T, b1.reshape(1, D))
    p2 = _SEG(h, srcp, dstp)
    if isinstance(p2, (list, tuple)):
        p2 = p2[0]
    out = _COMBINE_SIGMOID(p2, p2, cnt, h, W2_l.T, W2_r.T, b2.reshape(1, D))
    return out[:N_NODES]
